# baseline (speedup 1.0000x reference)
import jax
import jax.numpy as jnp
from jax import lax
from jax.experimental import pallas as pl
from jax.experimental.pallas import tpu as pltpu
from jax.experimental.pallas import tpu_sc as plsc

_N = 100
_PAD = 128
_NUM_IDX = 25


def _rotl(x, r):
    return (x << jnp.uint32(r)) | (x >> jnp.uint32(32 - r))


def _threefry2x32(k0, k1, x0, x1):
    ks = [k0, k1, k0 ^ k1 ^ jnp.uint32(0x1BD11BDA)]
    x0 = x0 + ks[0]
    x1 = x1 + ks[1]
    rotations = ((13, 15, 26, 6), (17, 29, 16, 24))
    for i in range(5):
        for r in rotations[i % 2]:
            x0 = x0 + x1
            x1 = _rotl(x1, r)
            x1 = x0 ^ x1
        x0 = x0 + ks[(i + 1) % 3]
        x1 = x1 + ks[(i + 2) % 3] + jnp.uint32(i + 1)
    return x0, x1


def _mask_body(out_ref, mask_ref):
    c = lax.axis_index("c")
    s = lax.axis_index("s")

    @pl.when(jnp.logical_and(c == 0, s == 0))
    def _():
        lane = lax.iota(jnp.int32, 16)
        zero_u = jnp.zeros((16,), jnp.uint32)

        one_u = zero_u + jnp.uint32(1)
        k0, k1 = _threefry2x32(zero_u, zero_u, zero_u, one_u)

        lo_a = lane.astype(jnp.uint32)
        lo_b = lo_a + jnp.uint32(16)
        a0, a1 = _threefry2x32(k0, k1, zero_u, lo_a)
        b0, b1 = _threefry2x32(k0, k1, zero_u, lo_b)
        bits_a = a0 ^ a1
        bits_b = b0 ^ b1

        def to_index(bits):
            f = lax.bitcast_convert_type(
                (bits >> jnp.uint32(9)) | jnp.uint32(0x3F800000), jnp.float32)
            u = f - jnp.float32(1.0)
            return (u * jnp.float32(_N)).astype(jnp.int32)

        idx_a = to_index(bits_a)
        idx_b = to_index(bits_b)

        zeros_v = jnp.zeros((16,), jnp.int32)
        for j in range(_PAD // 16):
            mask_ref[pl.ds(j * 16, 16)] = zeros_v
        ones_v = zeros_v + 1
        plsc.store_scatter(mask_ref, [idx_a], ones_v)
        plsc.store_scatter(mask_ref, [idx_b], ones_v,
                           mask=lane < (_NUM_IDX - 16))
        pltpu.sync_copy(mask_ref, out_ref)


def kernel(x):
    del x
    run = pl.kernel(
        _mask_body,
        out_type=jax.ShapeDtypeStruct((_PAD,), jnp.int32),
        mesh=plsc.VectorSubcoreMesh(core_axis_name="c", subcore_axis_name="s"),
        scratch_types=[pltpu.VMEM((_PAD,), jnp.int32)],
        compiler_params=pltpu.CompilerParams(needs_layout_passes=False),
    )
    out = run()
    return out[:_N].astype(jnp.bool_)


# speedup vs baseline: 1.1006x; 1.1006x over previous
import jax
import jax.numpy as jnp
from jax import lax
from jax.experimental import pallas as pl
from jax.experimental.pallas import tpu as pltpu
from jax.experimental.pallas import tpu_sc as plsc

_N = 100
_PAD = 128
_NUM_IDX = 25


def _rotl(x, r):
    return (x << jnp.uint32(r)) | (x >> jnp.uint32(32 - r))


def _threefry2x32(k0, k1, x0, x1):
    ks = [k0, k1, k0 ^ k1 ^ jnp.uint32(0x1BD11BDA)]
    x0 = x0 + ks[0]
    x1 = x1 + ks[1]
    rotations = ((13, 15, 26, 6), (17, 29, 16, 24))
    for i in range(5):
        for r in rotations[i % 2]:
            x0 = x0 + x1
            x1 = _rotl(x1, r)
            x1 = x0 ^ x1
        x0 = x0 + ks[(i + 1) % 3]
        x1 = x1 + ks[(i + 2) % 3] + jnp.uint32(i + 1)
    return x0, x1


def _mask_body(out_ref, mask_ref):
    if True:
        lane = lax.iota(jnp.int32, 16)
        zero_u = jnp.zeros((16,), jnp.uint32)

        one_u = zero_u + jnp.uint32(1)
        k0, k1 = _threefry2x32(zero_u, zero_u, zero_u, one_u)

        lo_a = lane.astype(jnp.uint32)
        lo_b = lo_a + jnp.uint32(16)
        a0, a1 = _threefry2x32(k0, k1, zero_u, lo_a)
        b0, b1 = _threefry2x32(k0, k1, zero_u, lo_b)
        bits_a = a0 ^ a1
        bits_b = b0 ^ b1

        def to_index(bits):
            f = lax.bitcast_convert_type(
                (bits >> jnp.uint32(9)) | jnp.uint32(0x3F800000), jnp.float32)
            u = f - jnp.float32(1.0)
            return (u * jnp.float32(_N)).astype(jnp.int32)

        idx_a = to_index(bits_a)
        idx_b = to_index(bits_b)

        zeros_v = jnp.zeros((16,), jnp.int32)
        for j in range(_PAD // 16):
            mask_ref[pl.ds(j * 16, 16)] = zeros_v
        ones_v = zeros_v + 1
        plsc.store_scatter(mask_ref, [idx_a], ones_v)
        plsc.store_scatter(mask_ref, [idx_b], ones_v,
                           mask=lane < (_NUM_IDX - 16))
        pltpu.sync_copy(mask_ref, out_ref)


def kernel(x):
    del x
    run = pl.kernel(
        _mask_body,
        out_type=jax.ShapeDtypeStruct((_PAD,), jnp.int32),
        mesh=plsc.VectorSubcoreMesh(core_axis_name="c", subcore_axis_name="s",
                                    num_cores=1, num_subcores=1),
        scratch_types=[pltpu.VMEM((_PAD,), jnp.int32)],
        compiler_params=pltpu.CompilerParams(needs_layout_passes=False),
    )
    out = run()
    return out[:_N].astype(jnp.bool_)


# speedup vs baseline: 10.6597x; 9.6858x over previous
import jax
import jax.numpy as jnp
from jax import lax
from jax.experimental import pallas as pl
from jax.experimental.pallas import tpu as pltpu

_N = 100
_NUM_IDX = 25
_R = 32


def _rotl(x, r):
    return (x << jnp.uint32(r)) | (x >> jnp.uint32(32 - r))


def _threefry2x32(k0, k1, x0, x1):
    ks = [k0, k1, k0 ^ k1 ^ jnp.uint32(0x1BD11BDA)]
    x0 = x0 + ks[0]
    x1 = x1 + ks[1]
    rotations = ((13, 15, 26, 6), (17, 29, 16, 24))
    for i in range(5):
        for r in rotations[i % 2]:
            x0 = x0 + x1
            x1 = _rotl(x1, r)
            x1 = x0 ^ x1
        x0 = x0 + ks[(i + 1) % 3]
        x1 = x1 + ks[(i + 2) % 3] + jnp.uint32(i + 1)
    return x0, x1


def _mask_body(out_ref):
    row = lax.broadcasted_iota(jnp.int32, (_R, 128), 0)
    zero_u = jnp.zeros((_R, 128), jnp.uint32)

    k0, k1 = _threefry2x32(zero_u, zero_u, zero_u, zero_u + jnp.uint32(1))

    b0, b1 = _threefry2x32(k0, k1, zero_u, row.astype(jnp.uint32))
    bits = b0 ^ b1

    f = lax.bitcast_convert_type(
        (bits >> jnp.uint32(9)) | jnp.uint32(0x3F800000), jnp.float32)
    idx = ((f - jnp.float32(1.0)) * jnp.float32(_N)).astype(jnp.int32)

    col = lax.broadcasted_iota(jnp.int32, (_R, 128), 1)
    hit = jnp.logical_and(idx == col, row < _NUM_IDX)
    mask = jnp.any(hit, axis=0)
    out_ref[...] = mask[:_N]


def kernel(x):
    del x
    return pl.pallas_call(
        _mask_body,
        out_shape=jax.ShapeDtypeStruct((_N,), jnp.bool_),
    )()
